# baseline (speedup 1.0000x reference)
import functools

import jax
import jax.numpy as jnp
from jax import lax
from jax.experimental import pallas as pl
from jax.experimental.pallas import tpu as pltpu
from jax.experimental.pallas import tpu_sc as plsc

EMBED_DIM = 64
CHUNK = 128


@functools.lru_cache(maxsize=None)
def _build(n_total):
    info = plsc.get_sparse_core_info()
    nc, ns = info.num_cores, info.num_subcores
    nw = nc * ns
    per_w = n_total // nw
    assert per_w * nw == n_total and per_w % CHUNK == 0
    n_chunks = per_w // CHUNK

    mesh = plsc.VectorSubcoreMesh(core_axis_name="c", subcore_axis_name="s")

    @functools.partial(
        pl.kernel,
        out_type=jax.ShapeDtypeStruct((n_total, EMBED_DIM), jnp.float32),
        mesh=mesh,
        scratch_types=[
            pltpu.VMEM((n_chunks, CHUNK), jnp.int32),
            pltpu.VMEM((CHUNK, EMBED_DIM), jnp.float32),
            pltpu.SemaphoreType.DMA,
        ],
        compiler_params=pltpu.CompilerParams(use_tc_tiling_on_sc=False),
    )
    def gather_kernel(idx_hbm, table_hbm, out_hbm, idx_v, rows, gsem):
        wid = lax.axis_index("s") * nc + lax.axis_index("c")
        base = wid * per_w

        pltpu.sync_copy(idx_hbm.at[wid], idx_v)

        def step(j, carry):
            pltpu.async_copy(table_hbm.at[idx_v.at[j]], rows, gsem).wait()
            pltpu.sync_copy(rows, out_hbm.at[pl.ds(base + j * CHUNK, CHUNK)])
            return carry

        lax.fori_loop(0, n_chunks, step, 0)

    return gather_kernel, nw, n_chunks


def kernel(indices, embeddings):
    batch, hist = indices.shape
    n_total = batch * hist
    run, nw, n_chunks = _build(n_total)
    idx3 = indices.reshape(nw, n_chunks, CHUNK)
    out = run(idx3, embeddings)
    return out.reshape(batch, hist, EMBED_DIM)


# speedup vs baseline: 1.1155x; 1.1155x over previous
import functools

import jax
import jax.numpy as jnp
from jax import lax
from jax.experimental import pallas as pl
from jax.experimental.pallas import tpu as pltpu
from jax.experimental.pallas import tpu_sc as plsc

EMBED_DIM = 64
CHUNK = 128
K = 4
NBUF = 2


@functools.lru_cache(maxsize=None)
def _build(n_total):
    info = plsc.get_sparse_core_info()
    nc, ns = info.num_cores, info.num_subcores
    nw = nc * ns
    per_w = n_total // nw
    group = K * CHUNK
    assert per_w * nw == n_total and per_w % group == 0
    n_chunks = per_w // CHUNK
    n_groups = per_w // group
    assert n_groups % NBUF == 0

    mesh = plsc.VectorSubcoreMesh(core_axis_name="c", subcore_axis_name="s")

    @functools.partial(
        pl.kernel,
        out_type=jax.ShapeDtypeStruct((n_total, EMBED_DIM), jnp.float32),
        mesh=mesh,
        scratch_types=[
            pltpu.VMEM((n_chunks, CHUNK), jnp.int32),
            [pltpu.VMEM((group, EMBED_DIM), jnp.float32) for _ in range(NBUF)],
            [pltpu.SemaphoreType.DMA for _ in range(NBUF)],
            [pltpu.SemaphoreType.DMA for _ in range(NBUF)],
        ],
        compiler_params=pltpu.CompilerParams(use_tc_tiling_on_sc=False),
    )
    def gather_kernel(idx_hbm, table_hbm, out_hbm, idx_v, rows, gsem, wsem):
        wid = lax.axis_index("s") * nc + lax.axis_index("c")
        base = wid * per_w

        pltpu.sync_copy(idx_hbm.at[wid], idx_v)

        def fire(g, b):
            for t in range(K):
                pltpu.async_copy(
                    table_hbm.at[idx_v.at[g * K + t]],
                    rows[b].at[pl.ds(t * CHUNK, CHUNK)],
                    gsem[b])

        def drain_gathers(g, b):
            for t in range(K):
                pltpu.make_async_copy(
                    table_hbm.at[idx_v.at[g * K + t]],
                    rows[b].at[pl.ds(t * CHUNK, CHUNK)],
                    gsem[b]).wait()

        def start_write(g, b):
            pltpu.async_copy(
                rows[b], out_hbm.at[pl.ds(base + g * group, group)], wsem[b])

        def wait_write(g, b):
            pltpu.make_async_copy(
                rows[b], out_hbm.at[pl.ds(base + g * group, group)],
                wsem[b]).wait()

        fire(0, 0)

        @pl.loop(0, n_groups, step=NBUF)
        def _(g0):
            for b in range(NBUF):
                g = g0 + b
                drain_gathers(g, b)
                nb = (b + 1) % NBUF

                @pl.when(g + 1 < n_groups)
                def _():
                    @pl.when(g + 1 >= NBUF)
                    def _():
                        wait_write(g + 1 - NBUF, nb)
                    fire(g + 1, nb)

                start_write(g, b)

        for b in range(NBUF):
            wait_write(n_groups - NBUF + b, b)

    return gather_kernel, nw, n_chunks


def kernel(indices, embeddings):
    batch, hist = indices.shape
    n_total = batch * hist
    run, nw, n_chunks = _build(n_total)
    idx3 = indices.reshape(nw, n_chunks, CHUNK)
    out = run(idx3, embeddings)
    return out.reshape(batch, hist, EMBED_DIM)
